# baseline (speedup 1.0000x reference)
import jax
import jax.numpy as jnp
from jax.experimental import pallas as pl

_BLOCK_B = 512


def _zero_fill(o_ref):
    o_ref[...] = jnp.zeros(o_ref.shape, o_ref.dtype)


def kernel(x, embedding):
    B, S = x.shape
    D = embedding.shape[1]
    dtype = embedding.dtype

    cols = S * D
    block_b = _BLOCK_B if B % _BLOCK_B == 0 else B
    out = pl.pallas_call(
        _zero_fill,
        grid=(B // block_b,),
        out_specs=pl.BlockSpec((block_b, cols), lambda i: (i, 0)),
        out_shape=jax.ShapeDtypeStruct((B, cols), dtype),
    )()
    return out.reshape(B, S, D)


# speedup vs baseline: 1.0038x; 1.0038x over previous
import jax
import jax.numpy as jnp
from jax.experimental import pallas as pl
from jax.experimental.pallas import tpu as pltpu

_ROWS = 256


def _make_fill(n_copies, rows):
    def _fill(o_ref, z_ref, sems):
        z_ref[...] = jnp.zeros(z_ref.shape, z_ref.dtype)
        for i in range(n_copies):
            pltpu.make_async_copy(
                z_ref, o_ref.at[pl.ds(i * rows, rows), :], sems.at[i]
            ).start()
        for i in range(n_copies):
            pltpu.make_async_copy(
                z_ref, o_ref.at[pl.ds(i * rows, rows), :], sems.at[i]
            ).wait()

    return _fill


def kernel(x, embedding):
    B, S = x.shape
    D = embedding.shape[1]
    dtype = embedding.dtype

    cols = S * D
    rows = _ROWS if B % _ROWS == 0 else B
    n_copies = B // rows
    out = pl.pallas_call(
        _make_fill(n_copies, rows),
        out_specs=pl.BlockSpec(memory_space=pltpu.MemorySpace.HBM),
        out_shape=jax.ShapeDtypeStruct((B, cols), dtype),
        scratch_shapes=[
            pltpu.VMEM((rows, cols), dtype),
            pltpu.SemaphoreType.DMA((n_copies,)),
        ],
    )()
    return out.reshape(B, S, D)


# speedup vs baseline: 1.0177x; 1.0138x over previous
import jax
import jax.numpy as jnp
from jax.experimental import pallas as pl
from jax.experimental.pallas import tpu as pltpu

_BLOCK_B = 256


def _fill(o_ref):
    o_ref[...] = jnp.zeros(o_ref.shape, o_ref.dtype)


def kernel(x, embedding):
    B, S = x.shape
    D = embedding.shape[1]
    dtype = embedding.dtype

    cols = S * D
    block_b = _BLOCK_B if B % _BLOCK_B == 0 else B
    out = pl.pallas_call(
        _fill,
        grid=(B // block_b,),
        out_specs=pl.BlockSpec((block_b, cols), lambda i: (i, 0)),
        out_shape=jax.ShapeDtypeStruct((B, cols), dtype),
        compiler_params=pltpu.CompilerParams(
            dimension_semantics=("parallel",),
        ),
    )()
    return out.reshape(B, S, D)
